# baseline (speedup 1.0000x reference)
import functools
import jax
import jax.numpy as jnp
from jax.experimental import pallas as pl

B = 32
IN_F = 512
OUT_F = 4096
TOP_K = 64
Q = OUT_F // B


def _s_build_kernel(inp_ref, psn_ref, s_ref):
    row = psn_ref[0, 0, :]
    iota_j = jax.lax.broadcasted_iota(jnp.int32, (IN_F, IN_F), 0)
    ohT = (iota_j == row[None, :]).astype(jnp.bfloat16)
    s = jax.lax.dot_general(inp_ref[...], ohT, (((1,), (1,)), ((), ())),
                            preferred_element_type=jnp.float32)
    s_ref[0, :, 0, :] = s


def _mm_topk_kernel(w_ref, s_ref, bias_ref, vals_ref, idx_ref):
    s = s_ref[:, 0, 0, :]
    w32 = w_ref[...].astype(jnp.float32)
    r = jax.lax.dot_general(w32, s, (((1,), (1,)), ((), ())),
                            precision=jax.lax.Precision.HIGHEST,
                            preferred_element_type=jnp.float32)
    r = r + bias_ref[...]
    posidx = (jax.lax.broadcasted_iota(jnp.int32, (Q, B), 0) * B
              + jax.lax.broadcasted_iota(jnp.int32, (Q, B), 1))
    kiota = jax.lax.broadcasted_iota(jnp.int32, (1, TOP_K), 1)
    cur = r
    vals_acc = jnp.zeros((1, TOP_K), jnp.float32)
    idx_acc = jnp.zeros((1, TOP_K), jnp.int32)
    for k in range(TOP_K):
        m = jnp.max(cur)
        sel = jnp.min(jnp.where(cur == m, posidx, OUT_F))
        vals_acc = jnp.where(kiota == k, m, vals_acc)
        idx_acc = jnp.where(kiota == k, sel, idx_acc)
        cur = jnp.where(posidx == sel, -jnp.inf, cur)
    vals_ref[0, :, :] = vals_acc
    idx_ref[0, :, :] = idx_acc


@jax.jit
def kernel(input, previously_selected_nodes, weight, bias):
    psn = previously_selected_nodes.astype(jnp.int32).reshape(B, 1, IN_F)
    inp = input.astype(jnp.bfloat16)
    wgt = weight.astype(jnp.bfloat16)

    s = pl.pallas_call(
        _s_build_kernel,
        grid=(B,),
        in_specs=[
            pl.BlockSpec((B, IN_F), lambda b: (0, 0)),
            pl.BlockSpec((1, 1, IN_F), lambda b: (b, 0, 0)),
        ],
        out_specs=pl.BlockSpec((1, B, 1, IN_F), lambda b: (b, 0, 0, 0)),
        out_shape=jax.ShapeDtypeStruct((B, B, 1, IN_F), jnp.float32),
    )(inp, psn)

    bias2d = bias.reshape(Q, B)
    vals, idx = pl.pallas_call(
        _mm_topk_kernel,
        grid=(B,),
        in_specs=[
            pl.BlockSpec((Q, IN_F), lambda c: (c, 0)),
            pl.BlockSpec((B, 1, 1, IN_F), lambda c: (0, c, 0, 0)),
            pl.BlockSpec((Q, B), lambda c: (0, 0)),
        ],
        out_specs=[
            pl.BlockSpec((1, 1, TOP_K), lambda c: (c, 0, 0)),
            pl.BlockSpec((1, 1, TOP_K), lambda c: (c, 0, 0)),
        ],
        out_shape=[
            jax.ShapeDtypeStruct((B, 1, TOP_K), jnp.float32),
            jax.ShapeDtypeStruct((B, 1, TOP_K), jnp.int32),
        ],
    )(wgt, s, bias2d)

    return vals.reshape(B, TOP_K), idx.reshape(B, TOP_K)


# speedup vs baseline: 9.8032x; 9.8032x over previous
import jax
import jax.numpy as jnp
from jax.experimental import pallas as pl

B = 32
IN_F = 512
OUT_F = 4096
TOP_K = 64
Q = OUT_F // B


def _s_build_kernel(inp_ref, psn_ref, s_ref):
    row = psn_ref[0, 0, :]
    iota_j = jax.lax.broadcasted_iota(jnp.int32, (IN_F, IN_F), 0)
    ohT = (iota_j == row[None, :]).astype(jnp.bfloat16)
    s = jax.lax.dot_general(inp_ref[...], ohT, (((1,), (1,)), ((), ())),
                            preferred_element_type=jnp.float32)
    s_ref[0, :, 0, :] = s


def _mm_kernel(w_ref, s_ref, bias_ref, out_ref):
    s = s_ref[:, 0, 0, :]
    w32 = w_ref[...].astype(jnp.float32)
    rt = jax.lax.dot_general(s, w32, (((1,), (1,)), ((), ())),
                             precision=jax.lax.Precision.HIGHEST,
                             preferred_element_type=jnp.float32)
    out_ref[0] = rt + bias_ref[...]


def _topk_kernel(res_ref, vals_ref, idx_ref):
    cur = res_ref[...]
    opp = jax.lax.broadcasted_iota(jnp.int32, (B, OUT_F), 1)
    posidx = (opp % Q) * B + opp // Q
    kio = jax.lax.broadcasted_iota(jnp.int32, (B, TOP_K), 1)
    vals_acc = jnp.zeros((B, TOP_K), jnp.float32)
    idx_acc = jnp.zeros((B, TOP_K), jnp.int32)
    neg_inf = jnp.float32(-jnp.inf)
    for k in range(TOP_K):
        m = jnp.max(cur, axis=1, keepdims=True)
        sel = jnp.min(jnp.where(cur == m, posidx, OUT_F), axis=1, keepdims=True)
        vals_acc = jnp.where(kio == k, m, vals_acc)
        idx_acc = jnp.where(kio == k, sel, idx_acc)
        cur = jnp.where(posidx == sel, neg_inf, cur)
    vals_ref[...] = vals_acc
    idx_ref[...] = idx_acc


@jax.jit
def kernel(input, previously_selected_nodes, weight, bias):
    psn = previously_selected_nodes.astype(jnp.int32).reshape(B, 1, IN_F)
    inp = input.astype(jnp.bfloat16)
    wgt = weight.astype(jnp.bfloat16)

    s = pl.pallas_call(
        _s_build_kernel,
        grid=(B,),
        in_specs=[
            pl.BlockSpec((B, IN_F), lambda b: (0, 0)),
            pl.BlockSpec((1, 1, IN_F), lambda b: (b, 0, 0)),
        ],
        out_specs=pl.BlockSpec((1, B, 1, IN_F), lambda b: (b, 0, 0, 0)),
        out_shape=jax.ShapeDtypeStruct((B, B, 1, IN_F), jnp.float32),
    )(inp, psn)

    bias_bq = bias.reshape(Q, B).T
    res = pl.pallas_call(
        _mm_kernel,
        grid=(B,),
        in_specs=[
            pl.BlockSpec((Q, IN_F), lambda c: (c, 0)),
            pl.BlockSpec((B, 1, 1, IN_F), lambda c: (0, c, 0, 0)),
            pl.BlockSpec((B, Q), lambda c: (0, 0)),
        ],
        out_specs=pl.BlockSpec((1, B, Q), lambda c: (c, 0, 0)),
        out_shape=jax.ShapeDtypeStruct((B, B, Q), jnp.float32),
    )(wgt, s, bias_bq)

    vals, idx = pl.pallas_call(
        _topk_kernel,
        in_specs=[pl.BlockSpec((B, OUT_F), lambda: (0, 0))],
        out_specs=[
            pl.BlockSpec((B, TOP_K), lambda: (0, 0)),
            pl.BlockSpec((B, TOP_K), lambda: (0, 0)),
        ],
        out_shape=[
            jax.ShapeDtypeStruct((B, TOP_K), jnp.float32),
            jax.ShapeDtypeStruct((B, TOP_K), jnp.int32),
        ],
    )(res.reshape(B, OUT_F))

    return vals, idx


# speedup vs baseline: 10.4768x; 1.0687x over previous
import jax
import jax.numpy as jnp
from jax.experimental import pallas as pl

B = 32
IN_F = 512
OUT_F = 4096
TOP_K = 64
Q = OUT_F // B


def _s_build_kernel(inp_ref, psn_ref, s_ref):
    row = psn_ref[0, 0, :]
    iota_j = jax.lax.broadcasted_iota(jnp.int32, (IN_F, IN_F), 0)
    ohT = (iota_j == row[None, :]).astype(jnp.bfloat16)
    inp_b = inp_ref[...].astype(jnp.bfloat16)
    s = jax.lax.dot_general(inp_b, ohT, (((1,), (1,)), ((), ())),
                            preferred_element_type=jnp.float32)
    s_ref[0, :, 0, :] = s


def _mm_kernel(w_ref, s_ref, bias_ref, out_ref):
    s = s_ref[:, 0, 0, :]
    w32 = w_ref[...].astype(jnp.bfloat16).astype(jnp.float32)
    rt = jax.lax.dot_general(w32, s, (((1,), (1,)), ((), ())),
                             precision=jax.lax.Precision.HIGHEST,
                             preferred_element_type=jnp.float32)
    out_ref[0] = rt + bias_ref[...]


def _topk_kernel(res_ref, vals_ref, idx_ref):
    cur = res_ref[...]
    posidx = jax.lax.broadcasted_iota(jnp.int32, (B, OUT_F), 1)
    kio = jax.lax.broadcasted_iota(jnp.int32, (B, TOP_K), 1)
    vals_acc = jnp.zeros((B, TOP_K), jnp.float32)
    idx_acc = jnp.zeros((B, TOP_K), jnp.int32)
    neg_inf = jnp.float32(-jnp.inf)
    for k in range(TOP_K):
        m = jnp.max(cur, axis=1, keepdims=True)
        sel = jnp.min(jnp.where(cur == m, posidx, OUT_F), axis=1, keepdims=True)
        vals_acc = jnp.where(kio == k, m, vals_acc)
        idx_acc = jnp.where(kio == k, sel, idx_acc)
        cur = jnp.where(posidx == sel, neg_inf, cur)
    vals_ref[...] = vals_acc
    idx_ref[...] = idx_acc


@jax.jit
def kernel(input, previously_selected_nodes, weight, bias):
    psn = previously_selected_nodes.astype(jnp.int32).reshape(B, 1, IN_F)

    s = pl.pallas_call(
        _s_build_kernel,
        grid=(B,),
        in_specs=[
            pl.BlockSpec((B, IN_F), lambda b: (0, 0)),
            pl.BlockSpec((1, 1, IN_F), lambda b: (b, 0, 0)),
        ],
        out_specs=pl.BlockSpec((1, B, 1, IN_F), lambda b: (b, 0, 0, 0)),
        out_shape=jax.ShapeDtypeStruct((B, B, 1, IN_F), jnp.float32),
    )(input, psn)

    res = pl.pallas_call(
        _mm_kernel,
        grid=(B,),
        in_specs=[
            pl.BlockSpec((Q, IN_F), lambda c: (c, 0)),
            pl.BlockSpec((B, 1, 1, IN_F), lambda c: (0, c, 0, 0)),
            pl.BlockSpec((Q, B), lambda c: (0, 0)),
        ],
        out_specs=pl.BlockSpec((1, Q, B), lambda c: (c, 0, 0)),
        out_shape=jax.ShapeDtypeStruct((B, Q, B), jnp.float32),
    )(weight, s, bias.reshape(Q, B))

    vals, idx = pl.pallas_call(
        _topk_kernel,
        in_specs=[pl.BlockSpec((B, OUT_F), lambda: (0, 0))],
        out_specs=[
            pl.BlockSpec((B, TOP_K), lambda: (0, 0)),
            pl.BlockSpec((B, TOP_K), lambda: (0, 0)),
        ],
        out_shape=[
            jax.ShapeDtypeStruct((B, TOP_K), jnp.float32),
            jax.ShapeDtypeStruct((B, TOP_K), jnp.int32),
        ],
    )(res.reshape(B, OUT_F))

    return vals, idx


# speedup vs baseline: 14.6885x; 1.4020x over previous
import jax
import jax.numpy as jnp
from jax.experimental import pallas as pl
from jax.experimental.pallas import tpu as pltpu

B = 32
IN_F = 512
OUT_F = 4096
TOP_K = 64
Q = OUT_F // B


def _fused_kernel(inp_ref, psn_ref, w_ref, bias_ref, vals_ref, idx_ref,
                  s_scr, res_scr):
    g = pl.program_id(0)

    @pl.when(g == 0)
    def _build_s():
        inp_b = inp_ref[...].astype(jnp.bfloat16)
        iota_j = jax.lax.broadcasted_iota(jnp.int32, (IN_F, IN_F), 0)
        for b in range(B):
            row = psn_ref[b, 0, :]
            ohT = (iota_j == row[None, :]).astype(jnp.bfloat16)
            s = jax.lax.dot_general(inp_b, ohT, (((1,), (1,)), ((), ())),
                                    preferred_element_type=jnp.float32)
            s_scr[:, b, :] = s

    @pl.when(jnp.logical_and(g >= 1, g <= B))
    def _mm():
        c = g - 1
        s = s_scr[c]
        w32 = w_ref[0].astype(jnp.bfloat16).astype(jnp.float32)
        rt = jax.lax.dot_general(s, w32, (((1,), (1,)), ((), ())),
                                 precision=jax.lax.Precision.HIGHEST,
                                 preferred_element_type=jnp.float32)
        res_scr[c] = rt + bias_ref[...]

    @pl.when(g == B + 1)
    def _topk():
        cur = res_scr[...]
        posidx = (jax.lax.broadcasted_iota(jnp.int32, (B, B, Q), 2) * B
                  + jax.lax.broadcasted_iota(jnp.int32, (B, B, Q), 1))
        kio = jax.lax.broadcasted_iota(jnp.int32, (B, TOP_K), 1)
        vals_acc = jnp.zeros((B, TOP_K), jnp.float32)
        idx_acc = jnp.zeros((B, TOP_K), jnp.int32)
        neg_inf = jnp.float32(-jnp.inf)
        for k in range(TOP_K):
            m = jnp.max(cur, axis=(1, 2), keepdims=True)
            sel = jnp.min(jnp.where(cur == m, posidx, OUT_F), axis=(1, 2),
                          keepdims=True)
            vals_acc = jnp.where(kio == k, m[:, :, 0], vals_acc)
            idx_acc = jnp.where(kio == k, sel[:, :, 0], idx_acc)
            cur = jnp.where(posidx == sel, neg_inf, cur)
        vals_ref[...] = vals_acc
        idx_ref[...] = idx_acc


@jax.jit
def kernel(input, previously_selected_nodes, weight, bias):
    psn = previously_selected_nodes.astype(jnp.int32).reshape(B, 1, IN_F)
    w3 = weight.reshape(B, Q, IN_F)
    bias_bq = bias.reshape(Q, B).T

    vals, idx = pl.pallas_call(
        _fused_kernel,
        grid=(B + 2,),
        in_specs=[
            pl.BlockSpec((B, IN_F), lambda g: (0, 0)),
            pl.BlockSpec((B, 1, IN_F), lambda g: (0, 0, 0)),
            pl.BlockSpec((1, Q, IN_F), lambda g: (jnp.maximum(g - 1, 0), 0, 0)),
            pl.BlockSpec((B, Q), lambda g: (0, 0)),
        ],
        out_specs=[
            pl.BlockSpec((B, TOP_K), lambda g: (0, 0)),
            pl.BlockSpec((B, TOP_K), lambda g: (0, 0)),
        ],
        out_shape=[
            jax.ShapeDtypeStruct((B, TOP_K), jnp.float32),
            jax.ShapeDtypeStruct((B, TOP_K), jnp.int32),
        ],
        scratch_shapes=[
            pltpu.VMEM((B, B, IN_F), jnp.float32),
            pltpu.VMEM((B, B, Q), jnp.float32),
        ],
    )(input, psn, w3, bias_bq)

    return vals, idx
